# Initial kernel scaffold; baseline (speedup 1.0000x reference)
import jax
import jax.numpy as jnp
from jax.experimental import pallas as pl


def kernel(obs, enc_W1, enc_b1, enc_W2, enc_b2, emb_weight, dec_W1, dec_b1, dec_W2, dec_b2):
    raise NotImplementedError("write your pallas kernel here")


# speedup vs baseline: 10.8122x; 10.8122x over previous
import jax
import jax.numpy as jnp
from jax import lax
from jax.experimental import pallas as pl
from jax.experimental.pallas import tpu as pltpu

OBS_DIM = 128
N_CODE_EACH = 128
CODE_DIM = 64
HIDDEN = 512
BATCH = 1024
BB = 128
DD = 16

F32 = jnp.float32
HI = jax.lax.Precision.HIGHEST


def _vqvae_kernel(obs_ref, w0_ref, w1r_ref, b1_ref, w2_ref, b2_ref,
                  codes_ref, dw1_ref, db1_ref, dw2_ref, db2_ref,
                  recon_ref, zet_ref, qt_ref, cn_ref):
    D, K, C, H = OBS_DIM, N_CODE_EACH, CODE_DIM, HIDDEN
    BF = jnp.bfloat16
    obs_t = obs_ref[...].T.astype(BF).astype(F32)
    w0 = w0_ref[...].astype(BF).astype(F32)
    b1 = b1_ref[...]
    w2 = w2_ref[...].astype(BF)
    b2 = b2_ref[...]

    @pl.when(pl.program_id(0) == 0)
    def _():
        codes = codes_ref[...]
        cn_ref[...] = jnp.sum(codes * codes, axis=-1)

    acc = db1_ref[...] * jnp.ones((BB, 1), F32)
    z_chunks = []
    q_chunks = []
    for d0 in range(0, D, DD):
        ot = obs_t[d0:d0 + DD]
        a = w1r_ref[d0:d0 + DD].astype(BF).astype(F32)
        h3 = jnp.maximum(
            (ot[:, :, None] * w0[None, :, :] + a[:, None, :])
            + b1[None, :, :], 0.0)
        z2 = jnp.dot(h3.reshape(DD * BB, H).astype(BF), w2,
                     preferred_element_type=F32) + b2
        z3 = z2.reshape(DD, BB, C)
        codes_c = codes_ref[d0:d0 + DD]
        dots = lax.dot_general(z3.astype(BF), codes_c.astype(BF),
                               (((2,), (2,)), ((0,), (0,))),
                               preferred_element_type=F32)
        zz = jnp.sum(z3 * z3, axis=-1, keepdims=True)
        d2 = (zz - 2.0 * dots) + cn_ref[d0:d0 + DD][:, None, :]
        idx = jnp.argmin(d2, axis=-1)
        onehot = (lax.broadcasted_iota(jnp.int32, (DD, BB, K), 2)
                  == idx[:, :, None]).astype(F32)
        q3 = lax.dot_general(onehot, codes_c, (((2,), (1,)), ((0,), (0,))),
                             precision=HI,
                             preferred_element_type=F32)
        t = lax.dot_general(q3, dw1_ref[d0:d0 + DD],
                            (((2,), (1,)), ((0,), (0,))),
                            preferred_element_type=F32)
        acc = acc + jnp.sum(t, axis=0)
        z_chunks.append(z3)
        q_chunks.append(q3)

    h1 = jnp.maximum(acc, 0.0)
    recon_ref[...] = jnp.dot(h1, dw2_ref[...],
                             preferred_element_type=F32) + db2_ref[...]
    z3_all = jnp.concatenate(z_chunks, axis=0)
    q3_all = jnp.concatenate(q_chunks, axis=0)
    zet_ref[...] = jnp.transpose(z3_all, (1, 2, 0))
    qt_ref[...] = jnp.transpose(q3_all, (1, 2, 0))


@jax.jit
def kernel(obs, enc_W1, enc_b1, enc_W2, enc_b2, emb_weight,
           dec_W1, dec_b1, dec_W2, dec_b2):
    D, K, C, H, B = OBS_DIM, N_CODE_EACH, CODE_DIM, HIDDEN, BATCH
    w0 = enc_W1[0:1]
    w1r = enc_W1[1:]
    codes = emb_weight.T.reshape(D, K, C)
    dw1 = dec_W1.reshape(D, C, H)

    grid = (B // BB,)
    full = lambda *s: pl.BlockSpec(s, lambda i: (0,) * len(s))
    recon, zet, qt = pl.pallas_call(
        _vqvae_kernel,
        grid=grid,
        in_specs=[
            pl.BlockSpec((BB, D), lambda i: (i, 0)),
            full(1, H), full(D, H), full(1, H),
            full(H, C), full(1, C),
            full(D, K, C),
            full(D, C, H), full(1, H),
            full(H, D), full(1, D),
        ],
        out_specs=[
            pl.BlockSpec((BB, D), lambda i: (i, 0)),
            pl.BlockSpec((BB, C, D), lambda i: (i, 0, 0)),
            pl.BlockSpec((BB, C, D), lambda i: (i, 0, 0)),
        ],
        out_shape=[
            jax.ShapeDtypeStruct((B, D), F32),
            jax.ShapeDtypeStruct((B, C, D), F32),
            jax.ShapeDtypeStruct((B, C, D), F32),
        ],
        scratch_shapes=[pltpu.VMEM((D, K), F32)],
    )(obs, w0, w1r, enc_b1.reshape(1, H), enc_W2, enc_b2.reshape(1, C),
      codes, dw1, dec_b1.reshape(1, H), dec_W2, dec_b2.reshape(1, D))

    return (recon, zet, qt)


# speedup vs baseline: 12.3480x; 1.1420x over previous
import jax
import jax.numpy as jnp
from jax import lax
from jax.experimental import pallas as pl
from jax.experimental.pallas import tpu as pltpu

OBS_DIM = 128
N_CODE_EACH = 128
CODE_DIM = 64
HIDDEN = 512
BATCH = 1024
BB = 128
DD = 16

F32 = jnp.float32
HI = jax.lax.Precision.HIGHEST


def _vqvae_kernel(obs_ref, w0_ref, w1r_ref, b1_ref, w2_ref, b2_ref,
                  codes_ref, dw1_ref, db1_ref, dw2_ref, db2_ref,
                  recon_ref, zet_ref, qt_ref, cn_ref):
    D, K, C, H = OBS_DIM, N_CODE_EACH, CODE_DIM, HIDDEN
    BF = jnp.bfloat16
    obs_t = obs_ref[...].T.astype(BF).astype(F32)
    w0 = w0_ref[...].astype(BF).astype(F32)
    b1 = b1_ref[...]
    w2 = w2_ref[...].astype(BF)
    b2 = b2_ref[...]

    @pl.when(pl.program_id(0) == 0)
    def _():
        codes = codes_ref[...]
        cn_ref[...] = jnp.sum(codes * codes, axis=-1)

    acc = db1_ref[...] * jnp.ones((BB, 1), F32)
    z_chunks = []
    q_chunks = []
    for d0 in range(0, D, DD):
        ot = obs_t[d0:d0 + DD]
        a = w1r_ref[d0:d0 + DD].astype(BF).astype(F32)
        h3 = jnp.maximum(
            (ot[:, :, None] * w0[None, :, :] + a[:, None, :])
            + b1[None, :, :], 0.0)
        z2 = jnp.dot(h3.reshape(DD * BB, H).astype(BF), w2,
                     preferred_element_type=F32) + b2
        z3 = z2.reshape(DD, BB, C)
        codes_c = codes_ref[d0:d0 + DD]
        dots = lax.dot_general(z3.astype(BF), codes_c.astype(BF),
                               (((2,), (2,)), ((0,), (0,))),
                               preferred_element_type=F32)
        d2 = cn_ref[d0:d0 + DD][:, None, :] - 2.0 * dots
        idx = jnp.argmin(d2, axis=-1)
        onehot = (lax.broadcasted_iota(jnp.int32, (DD, BB, K), 2)
                  == idx[:, :, None]).astype(BF)
        ch = codes_c.astype(BF)
        cl = (codes_c - ch.astype(F32)).astype(BF)
        dn = (((2,), (1,)), ((0,), (0,)))
        q3 = (lax.dot_general(onehot, ch, dn, preferred_element_type=F32)
              + lax.dot_general(onehot, cl, dn, preferred_element_type=F32))
        t = lax.dot_general(q3, dw1_ref[d0:d0 + DD],
                            (((2,), (1,)), ((0,), (0,))),
                            preferred_element_type=F32)
        acc = acc + jnp.sum(t, axis=0)
        z_chunks.append(z3)
        q_chunks.append(q3)

    h1 = jnp.maximum(acc, 0.0)
    recon_ref[...] = jnp.dot(h1, dw2_ref[...],
                             preferred_element_type=F32) + db2_ref[...]
    z3_all = jnp.concatenate(z_chunks, axis=0)
    q3_all = jnp.concatenate(q_chunks, axis=0)
    zet_ref[...] = jnp.transpose(z3_all, (1, 2, 0))
    qt_ref[...] = jnp.transpose(q3_all, (1, 2, 0))


@jax.jit
def kernel(obs, enc_W1, enc_b1, enc_W2, enc_b2, emb_weight,
           dec_W1, dec_b1, dec_W2, dec_b2):
    D, K, C, H, B = OBS_DIM, N_CODE_EACH, CODE_DIM, HIDDEN, BATCH
    w0 = enc_W1[0:1]
    w1r = enc_W1[1:]
    codes = emb_weight.T.reshape(D, K, C)
    dw1 = dec_W1.reshape(D, C, H)

    grid = (B // BB,)
    full = lambda *s: pl.BlockSpec(s, lambda i: (0,) * len(s))
    recon, zet, qt = pl.pallas_call(
        _vqvae_kernel,
        grid=grid,
        in_specs=[
            pl.BlockSpec((BB, D), lambda i: (i, 0)),
            full(1, H), full(D, H), full(1, H),
            full(H, C), full(1, C),
            full(D, K, C),
            full(D, C, H), full(1, H),
            full(H, D), full(1, D),
        ],
        out_specs=[
            pl.BlockSpec((BB, D), lambda i: (i, 0)),
            pl.BlockSpec((BB, C, D), lambda i: (i, 0, 0)),
            pl.BlockSpec((BB, C, D), lambda i: (i, 0, 0)),
        ],
        out_shape=[
            jax.ShapeDtypeStruct((B, D), F32),
            jax.ShapeDtypeStruct((B, C, D), F32),
            jax.ShapeDtypeStruct((B, C, D), F32),
        ],
        scratch_shapes=[pltpu.VMEM((D, K), F32)],
    )(obs, w0, w1r, enc_b1.reshape(1, H), enc_W2, enc_b2.reshape(1, C),
      codes, dw1, dec_b1.reshape(1, H), dec_W2, dec_b2.reshape(1, D))

    return (recon, zet, qt)
